# Initial kernel scaffold; baseline (speedup 1.0000x reference)
import jax
import jax.numpy as jnp
from jax.experimental import pallas as pl


def kernel(x, edge_index, W_in, b_in, W_out, b_out):
    raise NotImplementedError("write your pallas kernel here")


# speedup vs baseline: 4.9684x; 4.9684x over previous
import functools

import jax
import jax.numpy as jnp
from jax import lax
from jax.experimental import pallas as pl
from jax.experimental.pallas import tpu as pltpu
from jax.experimental.pallas import tpu_sc as plsc

N = 10000
E = 320000
D_IN = 128
D_HID = 128
D_OUT = 64
HOPS = 10

NC = 2
NS = 16
NP = 10240
RPT = NP // NS
DH = 64

EPW = E // (NC * NS)
EPT = E // NS
CHUNK = 128
NCHUNK = (EPT + CHUNK - 1) // CHUNK
EPT_P = NCHUNK * CHUNK
TRASH = NP
RCH = RPT // CHUNK

_mesh = plsc.VectorSubcoreMesh(core_axis_name="c", subcore_axis_name="s")


def _deg_body(dst_hbm, degp_out, dbuf, cnt, stage, tbuf, wacc):
    cid = lax.axis_index("c")
    sid = lax.axis_index("s")
    wid = cid * NS + sid
    zero = jnp.zeros((16,), jnp.float32)
    ones = jnp.full((16,), 1.0, jnp.float32)

    pltpu.sync_copy(dst_hbm.at[pl.ds(wid * EPW, EPW)], dbuf)

    @pl.loop(0, NP // 16)
    def _zero_cnt(i):
        cnt[pl.ds(i * 16, 16)] = zero

    @pl.loop(0, EPW // 16)
    def _count(i):
        d16 = dbuf[pl.ds(i * 16, 16)]
        plsc.addupdate_scatter(cnt, [d16], ones)

    pltpu.sync_copy(cnt, stage.at[sid])
    plsc.subcore_barrier()

    r0 = sid * RPT

    @pl.loop(0, RPT // 16)
    def _zero_w(i):
        wacc[pl.ds(i * 16, 16)] = zero

    for j in range(NS):
        pltpu.sync_copy(stage.at[j, pl.ds(r0, RPT)], tbuf)

        @pl.loop(0, RPT // 16)
        def _acc(i):
            wacc[pl.ds(i * 16, 16)] = wacc[pl.ds(i * 16, 16)] + tbuf[pl.ds(i * 16, 16)]

    pltpu.sync_copy(wacc, degp_out.at[cid, pl.ds(r0, RPT)])


_deg_call = pl.kernel(
    _deg_body,
    out_type=jax.ShapeDtypeStruct((NC, NP), jnp.float32),
    mesh=_mesh,
    scratch_types=[
        pltpu.VMEM((EPW,), jnp.int32),
        pltpu.VMEM((NP,), jnp.float32),
        pltpu.VMEM_SHARED((NS, NP), jnp.float32),
        pltpu.VMEM((RPT,), jnp.float32),
        pltpu.VMEM((RPT,), jnp.float32),
    ],
    compiler_params=pltpu.CompilerParams(needs_layout_passes=False),
)


def _hop_body(src_hbm, dst_hbm, degp, s0, s_work, acc_out,
              t_sh, sbufi, dbufi, gbuf, wbuf, zbuf, acc, invd,
              db0, db1, gsem):
    cid = lax.axis_index("c")
    sid = lax.axis_index("s")
    r0 = sid * RPT
    zero = jnp.zeros((16,), jnp.float32)
    s_view = s_work.at[cid]

    pltpu.sync_copy(degp.at[0, pl.ds(r0, RPT)], db0)
    pltpu.sync_copy(degp.at[1, pl.ds(r0, RPT)], db1)

    @pl.loop(0, RPT // 16)
    def _invd(i):
        d = jnp.maximum(db0[pl.ds(i * 16, 16)] + db1[pl.ds(i * 16, 16)], 1.0)
        invd[pl.ds(i * 16, 16)] = 1.0 / d

    @pl.loop(0, CHUNK)
    def _zz(r):
        for k in range(DH // 16):
            zbuf[r, pl.ds(k * 16, 16)] = zero

    for c in range(RCH):
        rg = r0 + c * CHUNK
        pltpu.sync_copy(zbuf, t_sh.at[pl.ds(rg, CHUNK)])
        pltpu.sync_copy(s0.at[cid, pl.ds(rg, CHUNK)], acc.at[pl.ds(c * CHUNK, CHUNK)])
        pltpu.sync_copy(acc.at[pl.ds(c * CHUNK, CHUNK)], s_view.at[pl.ds(rg, CHUNK)])

    @pl.when(sid == NS - 1)
    def _zt():
        pltpu.sync_copy(zbuf.at[pl.ds(0, 16)], t_sh.at[pl.ds(NP, 16)])

    plsc.subcore_barrier()

    def _hop(h, carry):
        def _chunk(j, c2):
            pltpu.sync_copy(src_hbm.at[sid, j], sbufi)
            pltpu.sync_copy(dst_hbm.at[sid, j], dbufi)
            pltpu.async_copy(s_view.at[sbufi], gbuf, gsem).wait()
            pltpu.sync_copy(gbuf, t_sh.at[dbufi], add=True)
            return c2

        lax.fori_loop(0, NCHUNK, _chunk, 0)
        plsc.subcore_barrier()

        for c in range(RCH):
            rg = r0 + c * CHUNK
            rl = c * CHUNK
            pltpu.sync_copy(t_sh.at[pl.ds(rg, CHUNK)], wbuf)
            pltpu.sync_copy(zbuf, t_sh.at[pl.ds(rg, CHUNK)])

            @pl.loop(0, CHUNK)
            def _scale(r):
                inv = invd[pl.ds(rl + r, 16)][0]
                for k in range(DH // 16):
                    v = wbuf[r, pl.ds(k * 16, 16)] * inv
                    wbuf[r, pl.ds(k * 16, 16)] = v
                    acc[rl + r, pl.ds(k * 16, 16)] = (
                        acc[rl + r, pl.ds(k * 16, 16)] + v)

            pltpu.sync_copy(wbuf, s_view.at[pl.ds(rg, CHUNK)])
        plsc.subcore_barrier()
        return carry

    lax.fori_loop(0, HOPS, _hop, 0)

    for c in range(RCH):
        rg = r0 + c * CHUNK
        pltpu.sync_copy(acc.at[pl.ds(c * CHUNK, CHUNK)],
                        acc_out.at[cid, pl.ds(rg, CHUNK)])


_hop_call = pl.kernel(
    _hop_body,
    out_type=(
        jax.ShapeDtypeStruct((NC, NP, DH), jnp.float32),
        jax.ShapeDtypeStruct((NC, NP, DH), jnp.float32),
    ),
    mesh=_mesh,
    scratch_types=[
        pltpu.VMEM_SHARED((NP + 16, DH), jnp.float32),
        pltpu.VMEM((CHUNK,), jnp.int32),
        pltpu.VMEM((CHUNK,), jnp.int32),
        pltpu.VMEM((CHUNK, DH), jnp.float32),
        pltpu.VMEM((CHUNK, DH), jnp.float32),
        pltpu.VMEM((CHUNK, DH), jnp.float32),
        pltpu.VMEM((RPT, DH), jnp.float32),
        pltpu.VMEM((RPT + 16,), jnp.float32),
        pltpu.VMEM((RPT,), jnp.float32),
        pltpu.VMEM((RPT,), jnp.float32),
        pltpu.SemaphoreType.DMA,
    ],
    compiler_params=pltpu.CompilerParams(
        needs_layout_passes=False, use_tc_tiling_on_sc=False),
)


def _mlp1_body(x_ref, w_ref, b_ref, d_ref, oa_ref, ob_ref):
    i = pl.program_id(0)
    h = jnp.dot(x_ref[...], w_ref[...], preferred_element_type=jnp.float32)
    h = jnp.maximum(h + b_ref[...], 0.0)
    d = jnp.maximum(d_ref[:, 0:1] + d_ref[:, 1:2], 1.0)
    s = h * lax.rsqrt(d)
    row = i * 640 + lax.broadcasted_iota(jnp.int32, (640, 1), 0)
    s = jnp.where(row < N, s, 0.0)
    oa_ref[...] = s[:, :DH]
    ob_ref[...] = s[:, DH:]


def _mlp1(x_p, w, b, degp_t):
    return pl.pallas_call(
        _mlp1_body,
        grid=(NP // 640,),
        in_specs=[
            pl.BlockSpec((640, D_IN), lambda i: (i, 0)),
            pl.BlockSpec((D_IN, D_HID), lambda i: (0, 0)),
            pl.BlockSpec((1, D_HID), lambda i: (0, 0)),
            pl.BlockSpec((640, 2), lambda i: (i, 0)),
        ],
        out_specs=(
            pl.BlockSpec((640, DH), lambda i: (i, 0)),
            pl.BlockSpec((640, DH), lambda i: (i, 0)),
        ),
        out_shape=(
            jax.ShapeDtypeStruct((NP, DH), jnp.float32),
            jax.ShapeDtypeStruct((NP, DH), jnp.float32),
        ),
    )(x_p, w, b, degp_t)


def _mlp2_body(a0_ref, a1_ref, d_ref, w_ref, b_ref, y_ref):
    d = jnp.maximum(d_ref[:, 0:1] + d_ref[:, 1:2], 1.0)
    scale = jnp.sqrt(d) * (1.0 / float(HOPS + 1))
    o = jnp.concatenate([a0_ref[...], a1_ref[...]], axis=1) * scale
    y = jnp.dot(o, w_ref[...], preferred_element_type=jnp.float32)
    y_ref[...] = y + b_ref[...]


def _mlp2(a0, a1, degp_t, w, b):
    blk = 1000
    return pl.pallas_call(
        _mlp2_body,
        grid=(N // blk,),
        in_specs=[
            pl.BlockSpec((blk, DH), lambda i: (i, 0)),
            pl.BlockSpec((blk, DH), lambda i: (i, 0)),
            pl.BlockSpec((blk, 2), lambda i: (i, 0)),
            pl.BlockSpec((D_HID, D_OUT), lambda i: (0, 0)),
            pl.BlockSpec((1, D_OUT), lambda i: (0, 0)),
        ],
        out_specs=pl.BlockSpec((blk, D_OUT), lambda i: (i, 0)),
        out_shape=jax.ShapeDtypeStruct((N, D_OUT), jnp.float32),
    )(a0, a1, degp_t, w, b)


@jax.jit
def kernel(x, edge_index, W_in, b_in, W_out, b_out):
    src = edge_index[0]
    dst = edge_index[1]
    pad = EPT_P * NS - E
    src_p = jnp.concatenate([src, jnp.zeros((pad,), jnp.int32)])
    dst_p = jnp.concatenate([dst, jnp.full((pad,), TRASH, jnp.int32)])
    src_p = src_p.reshape(NS, NCHUNK, CHUNK)
    dst_p = dst_p.reshape(NS, NCHUNK, CHUNK)

    degp = _deg_call(dst)
    degp_t = degp.T

    x_p = jnp.pad(x, ((0, NP - N), (0, 0)))
    s0a, s0b = _mlp1(x_p, W_in, b_in.reshape(1, D_HID), degp_t)
    s0 = jnp.stack([s0a, s0b], axis=0)

    _, acc = _hop_call(src_p, dst_p, degp, s0)

    y = _mlp2(acc[0, :N], acc[1, :N], degp_t[:N], W_out,
              b_out.reshape(1, D_OUT))
    return y


# speedup vs baseline: 13.4896x; 2.7151x over previous
import functools

import jax
import jax.numpy as jnp
from jax import lax
from jax.experimental import pallas as pl
from jax.experimental.pallas import tpu as pltpu
from jax.experimental.pallas import tpu_sc as plsc

N = 10000
E = 320000
D_IN = 128
D_HID = 128
D_OUT = 64
HOPS = 10

NC = 2
NS = 16
NP = 10240
RPT = NP // NS
DH = 64

EPW = E // (NC * NS)
EPT = E // NS
CHUNK = 128
NCHUNK = (EPT + CHUNK - 1) // CHUNK
EPT_P = NCHUNK * CHUNK
TRASH = NP
RCH = RPT // CHUNK

_mesh = plsc.VectorSubcoreMesh(core_axis_name="c", subcore_axis_name="s")


def _deg_body(dst_hbm, degp_out, dbuf, cnt, stage, tbuf, wacc):
    cid = lax.axis_index("c")
    sid = lax.axis_index("s")
    wid = cid * NS + sid
    zero = jnp.zeros((16,), jnp.float32)
    ones = jnp.full((16,), 1.0, jnp.float32)

    pltpu.sync_copy(dst_hbm.at[pl.ds(wid * EPW, EPW)], dbuf)

    @pl.loop(0, NP // 16)
    def _zero_cnt(i):
        cnt[pl.ds(i * 16, 16)] = zero

    @pl.loop(0, EPW // 16)
    def _count(i):
        d16 = dbuf[pl.ds(i * 16, 16)]
        plsc.addupdate_scatter(cnt, [d16], ones)

    pltpu.sync_copy(cnt, stage.at[sid])
    plsc.subcore_barrier()

    r0 = sid * RPT

    @pl.loop(0, RPT // 16)
    def _zero_w(i):
        wacc[pl.ds(i * 16, 16)] = zero

    for j in range(NS):
        pltpu.sync_copy(stage.at[j, pl.ds(r0, RPT)], tbuf)

        @pl.loop(0, RPT // 16)
        def _acc(i):
            wacc[pl.ds(i * 16, 16)] = wacc[pl.ds(i * 16, 16)] + tbuf[pl.ds(i * 16, 16)]

    pltpu.sync_copy(wacc, degp_out.at[cid, pl.ds(r0, RPT)])


_deg_call = pl.kernel(
    _deg_body,
    out_type=jax.ShapeDtypeStruct((NC, NP), jnp.float32),
    mesh=_mesh,
    scratch_types=[
        pltpu.VMEM((EPW,), jnp.int32),
        pltpu.VMEM((NP,), jnp.float32),
        pltpu.VMEM_SHARED((NS, NP), jnp.float32),
        pltpu.VMEM((RPT,), jnp.float32),
        pltpu.VMEM((RPT,), jnp.float32),
    ],
    compiler_params=pltpu.CompilerParams(needs_layout_passes=False),
)


NB = 3
NQ = 6


def _hop_body(src_hbm, dst_hbm, degp, s0, s_work, acc_out,
              t_sh, sb0, sb1, sb2, sb3, sb4, sb5,
              qb0, qb1, qb2, qb3, qb4, qb5,
              gb0, gb1, gb2, wbuf, zbuf, acc, invd, db0, db1,
              is0, is1, is2, is3, is4, is5,
              gs0, gs1, gs2, ss0, ss1, ss2):
    sb = [sb0, sb1, sb2, sb3, sb4, sb5]
    qb = [qb0, qb1, qb2, qb3, qb4, qb5]
    gb = [gb0, gb1, gb2]
    isem = [is0, is1, is2, is3, is4, is5]
    gsem = [gs0, gs1, gs2]
    ssem = [ss0, ss1, ss2]
    cid = lax.axis_index("c")
    sid = lax.axis_index("s")
    r0 = sid * RPT
    zero = jnp.zeros((16,), jnp.float32)
    s_view = s_work.at[cid]

    def issue_idx(j, q):
        pltpu.async_copy(src_hbm.at[sid, j], sb[q], isem[q])
        pltpu.async_copy(dst_hbm.at[sid, j], qb[q], isem[q])

    def wait_idx(j, q):
        pltpu.make_async_copy(src_hbm.at[sid, j], sb[q], isem[q]).wait()
        pltpu.make_async_copy(dst_hbm.at[sid, j], qb[q], isem[q]).wait()

    def issue_gather(b, q):
        pltpu.async_copy(s_view.at[sb[q]], gb[b], gsem[b])

    def wait_gather(b, q):
        pltpu.make_async_copy(s_view.at[sb[q]], gb[b], gsem[b]).wait()

    def issue_scatter(b, q):
        pltpu.async_copy(gb[b], t_sh.at[qb[q]], ssem[b], add=True)

    def wait_scatter(b, q):
        pltpu.make_async_copy(gb[b], t_sh.at[qb[q]], ssem[b]).wait()

    pltpu.sync_copy(degp.at[0, pl.ds(r0, RPT)], db0)
    pltpu.sync_copy(degp.at[1, pl.ds(r0, RPT)], db1)

    @pl.loop(0, RPT // 16)
    def _invd(i):
        d = jnp.maximum(db0[pl.ds(i * 16, 16)] + db1[pl.ds(i * 16, 16)], 1.0)
        invd[pl.ds(i * 16, 16)] = 1.0 / d

    @pl.loop(0, CHUNK)
    def _zz(r):
        for k in range(DH // 16):
            zbuf[r, pl.ds(k * 16, 16)] = zero

    for c in range(RCH):
        rg = r0 + c * CHUNK
        pltpu.sync_copy(zbuf, t_sh.at[pl.ds(rg, CHUNK)])
        pltpu.sync_copy(s0.at[cid, pl.ds(rg, CHUNK)], acc.at[pl.ds(c * CHUNK, CHUNK)])
        pltpu.sync_copy(acc.at[pl.ds(c * CHUNK, CHUNK)], s_view.at[pl.ds(rg, CHUNK)])

    @pl.when(sid == NS - 1)
    def _zt():
        pltpu.sync_copy(zbuf.at[pl.ds(0, 16)], t_sh.at[pl.ds(NP, 16)])

    plsc.subcore_barrier()

    def _hop(h, carry):
        issue_idx(0, 0)
        issue_idx(1, 1)
        wait_idx(0, 0)
        issue_gather(0, 0)

        @pl.loop(0, NCHUNK, step=NQ)
        def _blk(j0):
            for u in range(NQ):
                j = j0 + u
                b = u % NB
                b1 = (u + 1) % NB
                q1 = (u + 1) % NQ
                q2 = (u + 2) % NQ

                @pl.when(j + 2 < NCHUNK)
                def _():
                    issue_idx(j + 2, q2)

                @pl.when((j >= 2) & (j < NCHUNK + 2))
                def _():
                    wait_scatter(b1, q1)

                @pl.when(j + 1 < NCHUNK)
                def _():
                    wait_idx(j + 1, q1)
                    issue_gather(b1, q1)

                @pl.when(j < NCHUNK)
                def _():
                    wait_gather(b, u)
                    issue_scatter(b, u)

        plsc.subcore_barrier()

        for c in range(RCH):
            rg = r0 + c * CHUNK
            rl = c * CHUNK
            pltpu.sync_copy(t_sh.at[pl.ds(rg, CHUNK)], wbuf)
            pltpu.sync_copy(zbuf, t_sh.at[pl.ds(rg, CHUNK)])

            @pl.loop(0, CHUNK)
            def _scale(r):
                inv = invd[pl.ds(rl + r, 16)][0]
                for k in range(DH // 16):
                    v = wbuf[r, pl.ds(k * 16, 16)] * inv
                    wbuf[r, pl.ds(k * 16, 16)] = v
                    acc[rl + r, pl.ds(k * 16, 16)] = (
                        acc[rl + r, pl.ds(k * 16, 16)] + v)

            pltpu.sync_copy(wbuf, s_view.at[pl.ds(rg, CHUNK)])
        plsc.subcore_barrier()
        return carry

    lax.fori_loop(0, HOPS, _hop, 0)

    for c in range(RCH):
        rg = r0 + c * CHUNK
        pltpu.sync_copy(acc.at[pl.ds(c * CHUNK, CHUNK)],
                        acc_out.at[cid, pl.ds(rg, CHUNK)])


_hop_call = pl.kernel(
    _hop_body,
    out_type=(
        jax.ShapeDtypeStruct((NC, NP, DH), jnp.float32),
        jax.ShapeDtypeStruct((NC, NP, DH), jnp.float32),
    ),
    mesh=_mesh,
    scratch_types=(
        [pltpu.VMEM_SHARED((NP + 16, DH), jnp.float32)]
        + [pltpu.VMEM((CHUNK,), jnp.int32)] * NQ
        + [pltpu.VMEM((CHUNK,), jnp.int32)] * NQ
        + [pltpu.VMEM((CHUNK, DH), jnp.float32)] * NB
        + [
            pltpu.VMEM((CHUNK, DH), jnp.float32),
            pltpu.VMEM((CHUNK, DH), jnp.float32),
            pltpu.VMEM((RPT, DH), jnp.float32),
            pltpu.VMEM((RPT + 16,), jnp.float32),
            pltpu.VMEM((RPT,), jnp.float32),
            pltpu.VMEM((RPT,), jnp.float32),
        ]
        + [pltpu.SemaphoreType.DMA] * (NQ + NB + NB)
    ),
    compiler_params=pltpu.CompilerParams(
        needs_layout_passes=False, use_tc_tiling_on_sc=False),
)


def _mlp1_body(x_ref, w_ref, b_ref, d_ref, oa_ref, ob_ref):
    i = pl.program_id(0)
    h = jnp.dot(x_ref[...], w_ref[...], preferred_element_type=jnp.float32)
    h = jnp.maximum(h + b_ref[...], 0.0)
    d = jnp.maximum(d_ref[:, 0:1] + d_ref[:, 1:2], 1.0)
    s = h * lax.rsqrt(d)
    row = i * 640 + lax.broadcasted_iota(jnp.int32, (640, 1), 0)
    s = jnp.where(row < N, s, 0.0)
    oa_ref[...] = s[:, :DH]
    ob_ref[...] = s[:, DH:]


def _mlp1(x_p, w, b, degp_t):
    return pl.pallas_call(
        _mlp1_body,
        grid=(NP // 640,),
        in_specs=[
            pl.BlockSpec((640, D_IN), lambda i: (i, 0)),
            pl.BlockSpec((D_IN, D_HID), lambda i: (0, 0)),
            pl.BlockSpec((1, D_HID), lambda i: (0, 0)),
            pl.BlockSpec((640, 2), lambda i: (i, 0)),
        ],
        out_specs=(
            pl.BlockSpec((640, DH), lambda i: (i, 0)),
            pl.BlockSpec((640, DH), lambda i: (i, 0)),
        ),
        out_shape=(
            jax.ShapeDtypeStruct((NP, DH), jnp.float32),
            jax.ShapeDtypeStruct((NP, DH), jnp.float32),
        ),
    )(x_p, w, b, degp_t)


def _mlp2_body(a0_ref, a1_ref, d_ref, w_ref, b_ref, y_ref):
    d = jnp.maximum(d_ref[:, 0:1] + d_ref[:, 1:2], 1.0)
    scale = jnp.sqrt(d) * (1.0 / float(HOPS + 1))
    o = jnp.concatenate([a0_ref[...], a1_ref[...]], axis=1) * scale
    y = jnp.dot(o, w_ref[...], preferred_element_type=jnp.float32)
    y_ref[...] = y + b_ref[...]


def _mlp2(a0, a1, degp_t, w, b):
    blk = 1000
    return pl.pallas_call(
        _mlp2_body,
        grid=(N // blk,),
        in_specs=[
            pl.BlockSpec((blk, DH), lambda i: (i, 0)),
            pl.BlockSpec((blk, DH), lambda i: (i, 0)),
            pl.BlockSpec((blk, 2), lambda i: (i, 0)),
            pl.BlockSpec((D_HID, D_OUT), lambda i: (0, 0)),
            pl.BlockSpec((1, D_OUT), lambda i: (0, 0)),
        ],
        out_specs=pl.BlockSpec((blk, D_OUT), lambda i: (i, 0)),
        out_shape=jax.ShapeDtypeStruct((N, D_OUT), jnp.float32),
    )(a0, a1, degp_t, w, b)


@jax.jit
def kernel(x, edge_index, W_in, b_in, W_out, b_out):
    src = edge_index[0]
    dst = edge_index[1]
    pad = EPT_P * NS - E
    src_p = jnp.concatenate([src, jnp.zeros((pad,), jnp.int32)])
    dst_p = jnp.concatenate([dst, jnp.full((pad,), TRASH, jnp.int32)])
    src_p = src_p.reshape(NS, NCHUNK, CHUNK)
    dst_p = dst_p.reshape(NS, NCHUNK, CHUNK)

    degp = _deg_call(dst)
    degp_t = degp.T

    x_p = jnp.pad(x, ((0, NP - N), (0, 0)))
    s0a, s0b = _mlp1(x_p, W_in, b_in.reshape(1, D_HID), degp_t)
    s0 = jnp.stack([s0a, s0b], axis=0)

    _, acc = _hop_call(src_p, dst_p, degp, s0)

    y = _mlp2(acc[0, :N], acc[1, :N], degp_t[:N], W_out,
              b_out.reshape(1, D_OUT))
    return y


# speedup vs baseline: 13.9189x; 1.0318x over previous
import functools

import jax
import jax.numpy as jnp
from jax import lax
from jax.experimental import pallas as pl
from jax.experimental.pallas import tpu as pltpu
from jax.experimental.pallas import tpu_sc as plsc

N = 10000
E = 320000
D_IN = 128
D_HID = 128
D_OUT = 64
HOPS = 10

NC = 2
NS = 16
NP = 10240
RPT = NP // NS
DH = 64

EPW = E // (NC * NS)
EPT = E // NS
CHUNK = 128
NCHUNK = (EPT + CHUNK - 1) // CHUNK
EPT_P = NCHUNK * CHUNK
TRASH = NP
RCH = RPT // CHUNK

_mesh = plsc.VectorSubcoreMesh(core_axis_name="c", subcore_axis_name="s")


def _deg_body(dst_hbm, degp_out, dbuf, cnt, stage, tbuf, wacc):
    cid = lax.axis_index("c")
    sid = lax.axis_index("s")
    wid = cid * NS + sid
    zero = jnp.zeros((16,), jnp.float32)
    ones = jnp.full((16,), 1.0, jnp.float32)

    pltpu.sync_copy(dst_hbm.at[pl.ds(wid * EPW, EPW)], dbuf)

    @pl.loop(0, NP // 16)
    def _zero_cnt(i):
        cnt[pl.ds(i * 16, 16)] = zero

    @pl.loop(0, EPW // 16)
    def _count(i):
        d16 = dbuf[pl.ds(i * 16, 16)]
        plsc.addupdate_scatter(cnt, [d16], ones)

    pltpu.sync_copy(cnt, stage.at[sid])
    plsc.subcore_barrier()

    r0 = sid * RPT

    @pl.loop(0, RPT // 16)
    def _zero_w(i):
        wacc[pl.ds(i * 16, 16)] = zero

    for j in range(NS):
        pltpu.sync_copy(stage.at[j, pl.ds(r0, RPT)], tbuf)

        @pl.loop(0, RPT // 16)
        def _acc(i):
            wacc[pl.ds(i * 16, 16)] = wacc[pl.ds(i * 16, 16)] + tbuf[pl.ds(i * 16, 16)]

    pltpu.sync_copy(wacc, degp_out.at[cid, pl.ds(r0, RPT)])


_deg_call = pl.kernel(
    _deg_body,
    out_type=jax.ShapeDtypeStruct((NC, NP), jnp.float32),
    mesh=_mesh,
    scratch_types=[
        pltpu.VMEM((EPW,), jnp.int32),
        pltpu.VMEM((NP,), jnp.float32),
        pltpu.VMEM_SHARED((NS, NP), jnp.float32),
        pltpu.VMEM((RPT,), jnp.float32),
        pltpu.VMEM((RPT,), jnp.float32),
    ],
    compiler_params=pltpu.CompilerParams(needs_layout_passes=False),
)


NB = 4
NQ = 6
NU = 12


def _hop_body(src_hbm, dst_hbm, degp, s0, zrows, s_work, acc_out,
              t_sh, sb0, sb1, sb2, sb3, sb4, sb5,
              qb0, qb1, qb2, qb3, qb4, qb5,
              gb0, gb1, gb2, gb3, acc, invd, db0, db1,
              is0, is1, is2, is3, is4, is5,
              gs0, gs1, gs2, gs3, ss0, ss1, ss2, ss3,
              ws0, ws1, zs):
    sb = [sb0, sb1, sb2, sb3, sb4, sb5]
    qb = [qb0, qb1, qb2, qb3, qb4, qb5]
    gb = [gb0, gb1, gb2, gb3]
    isem = [is0, is1, is2, is3, is4, is5]
    gsem = [gs0, gs1, gs2, gs3]
    ssem = [ss0, ss1, ss2, ss3]
    wsem = [ws0, ws1]
    cid = lax.axis_index("c")
    sid = lax.axis_index("s")
    r0 = sid * RPT
    zero = jnp.zeros((16,), jnp.float32)
    s_view = s_work.at[cid]

    def issue_idx(j, q):
        pltpu.async_copy(src_hbm.at[sid, j], sb[q], isem[q])
        pltpu.async_copy(dst_hbm.at[sid, j], qb[q], isem[q])

    def wait_idx(j, q):
        pltpu.make_async_copy(src_hbm.at[sid, j], sb[q], isem[q]).wait()
        pltpu.make_async_copy(dst_hbm.at[sid, j], qb[q], isem[q]).wait()

    def issue_gather(b, q):
        pltpu.async_copy(s_view.at[sb[q]], gb[b], gsem[b])

    def wait_gather(b, q):
        pltpu.make_async_copy(s_view.at[sb[q]], gb[b], gsem[b]).wait()

    def issue_scatter(b, q):
        pltpu.async_copy(gb[b], t_sh.at[qb[q]], ssem[b], add=True)

    def wait_scatter(b, q):
        pltpu.make_async_copy(gb[b], t_sh.at[qb[q]], ssem[b]).wait()

    pltpu.sync_copy(degp.at[0, pl.ds(r0, RPT)], db0)
    pltpu.sync_copy(degp.at[1, pl.ds(r0, RPT)], db1)

    @pl.loop(0, RPT // 16)
    def _invd(i):
        d = jnp.maximum(db0[pl.ds(i * 16, 16)] + db1[pl.ds(i * 16, 16)], 1.0)
        invd[pl.ds(i * 16, 16)] = 1.0 / d

    for c in range(RCH):
        rg = r0 + c * CHUNK
        pltpu.sync_copy(zrows, t_sh.at[pl.ds(rg, CHUNK)])
        pltpu.sync_copy(s0.at[cid, pl.ds(rg, CHUNK)], acc.at[pl.ds(c * CHUNK, CHUNK)])
        pltpu.sync_copy(acc.at[pl.ds(c * CHUNK, CHUNK)], s_view.at[pl.ds(rg, CHUNK)])

    @pl.when(sid == NS - 1)
    def _zt():
        pltpu.sync_copy(zrows.at[pl.ds(0, 16)], t_sh.at[pl.ds(NP, 16)])

    plsc.subcore_barrier()

    def _hop(h, carry):
        issue_idx(0, 0)
        issue_idx(1, 1)
        issue_idx(2, 2)
        wait_idx(0, 0)
        issue_gather(0, 0)
        wait_idx(1, 1)
        issue_gather(1, 1)

        @pl.loop(0, NCHUNK, step=NU)
        def _blk(j0):
            for u in range(NU):
                j = j0 + u

                @pl.when(j + 3 < NCHUNK)
                def _():
                    issue_idx(j + 3, (u + 3) % NQ)

                @pl.when((j >= 2) & (j < NCHUNK + 2))
                def _():
                    wait_scatter((u + 2) % NB, (u + 4) % NQ)

                @pl.when(j + 2 < NCHUNK)
                def _():
                    wait_idx(j + 2, (u + 2) % NQ)
                    issue_gather((u + 2) % NB, (u + 2) % NQ)

                @pl.when(j < NCHUNK)
                def _():
                    wait_gather(u % NB, u % NQ)
                    issue_scatter(u % NB, u % NQ)

        plsc.subcore_barrier()

        for c in range(RCH):
            rg = r0 + c * CHUNK
            rl = c * CHUNK
            w = gb[c % 2]
            if c >= 2:
                pltpu.make_async_copy(
                    gb[c % 2],
                    s_view.at[pl.ds(r0 + (c - 2) * CHUNK, CHUNK)],
                    wsem[c % 2]).wait()
            pltpu.sync_copy(t_sh.at[pl.ds(rg, CHUNK)], w)
            pltpu.async_copy(zrows, t_sh.at[pl.ds(rg, CHUNK)], zs)

            @pl.loop(0, CHUNK)
            def _scale(r):
                inv = invd[pl.ds(rl + r, 16)][0]
                for k in range(DH // 16):
                    v = w[r, pl.ds(k * 16, 16)] * inv
                    w[r, pl.ds(k * 16, 16)] = v
                    acc[rl + r, pl.ds(k * 16, 16)] = (
                        acc[rl + r, pl.ds(k * 16, 16)] + v)

            pltpu.async_copy(w, s_view.at[pl.ds(rg, CHUNK)], wsem[c % 2])

        for c in (RCH - 2, RCH - 1):
            pltpu.make_async_copy(
                gb[c % 2], s_view.at[pl.ds(r0 + c * CHUNK, CHUNK)],
                wsem[c % 2]).wait()
        for c in range(RCH):
            pltpu.make_async_copy(
                zrows, t_sh.at[pl.ds(r0 + c * CHUNK, CHUNK)], zs).wait()
        plsc.subcore_barrier()
        return carry

    lax.fori_loop(0, HOPS, _hop, 0)

    for c in range(RCH):
        rg = r0 + c * CHUNK
        pltpu.sync_copy(acc.at[pl.ds(c * CHUNK, CHUNK)],
                        acc_out.at[cid, pl.ds(rg, CHUNK)])


_hop_call = pl.kernel(
    _hop_body,
    out_type=(
        jax.ShapeDtypeStruct((NC, NP, DH), jnp.float32),
        jax.ShapeDtypeStruct((NC, NP, DH), jnp.float32),
    ),
    mesh=_mesh,
    scratch_types=(
        [pltpu.VMEM_SHARED((NP + 16, DH), jnp.float32)]
        + [pltpu.VMEM((CHUNK,), jnp.int32)] * NQ
        + [pltpu.VMEM((CHUNK,), jnp.int32)] * NQ
        + [pltpu.VMEM((CHUNK, DH), jnp.float32)] * NB
        + [
            pltpu.VMEM((RPT, DH), jnp.float32),
            pltpu.VMEM((RPT + 16,), jnp.float32),
            pltpu.VMEM((RPT,), jnp.float32),
            pltpu.VMEM((RPT,), jnp.float32),
        ]
        + [pltpu.SemaphoreType.DMA] * (NQ + NB + NB + 3)
    ),
    compiler_params=pltpu.CompilerParams(
        needs_layout_passes=False, use_tc_tiling_on_sc=False),
)


def _mlp1_body(x_ref, w_ref, b_ref, d_ref, oa_ref, ob_ref):
    i = pl.program_id(0)
    h = jnp.dot(x_ref[...], w_ref[...], preferred_element_type=jnp.float32)
    h = jnp.maximum(h + b_ref[...], 0.0)
    d = jnp.maximum(d_ref[:, 0:1] + d_ref[:, 1:2], 1.0)
    s = h * lax.rsqrt(d)
    row = i * 640 + lax.broadcasted_iota(jnp.int32, (640, 1), 0)
    s = jnp.where(row < N, s, 0.0)
    oa_ref[...] = s[:, :DH]
    ob_ref[...] = s[:, DH:]


def _mlp1(x_p, w, b, degp_t):
    return pl.pallas_call(
        _mlp1_body,
        grid=(NP // 640,),
        in_specs=[
            pl.BlockSpec((640, D_IN), lambda i: (i, 0)),
            pl.BlockSpec((D_IN, D_HID), lambda i: (0, 0)),
            pl.BlockSpec((1, D_HID), lambda i: (0, 0)),
            pl.BlockSpec((640, 2), lambda i: (i, 0)),
        ],
        out_specs=(
            pl.BlockSpec((640, DH), lambda i: (i, 0)),
            pl.BlockSpec((640, DH), lambda i: (i, 0)),
        ),
        out_shape=(
            jax.ShapeDtypeStruct((NP, DH), jnp.float32),
            jax.ShapeDtypeStruct((NP, DH), jnp.float32),
        ),
    )(x_p, w, b, degp_t)


def _mlp2_body(a0_ref, a1_ref, d_ref, w_ref, b_ref, y_ref):
    d = jnp.maximum(d_ref[:, 0:1] + d_ref[:, 1:2], 1.0)
    scale = jnp.sqrt(d) * (1.0 / float(HOPS + 1))
    o = jnp.concatenate([a0_ref[...], a1_ref[...]], axis=1) * scale
    y = jnp.dot(o, w_ref[...], preferred_element_type=jnp.float32)
    y_ref[...] = y + b_ref[...]


def _mlp2(a0, a1, degp_t, w, b):
    blk = 1000
    return pl.pallas_call(
        _mlp2_body,
        grid=(N // blk,),
        in_specs=[
            pl.BlockSpec((blk, DH), lambda i: (i, 0)),
            pl.BlockSpec((blk, DH), lambda i: (i, 0)),
            pl.BlockSpec((blk, 2), lambda i: (i, 0)),
            pl.BlockSpec((D_HID, D_OUT), lambda i: (0, 0)),
            pl.BlockSpec((1, D_OUT), lambda i: (0, 0)),
        ],
        out_specs=pl.BlockSpec((blk, D_OUT), lambda i: (i, 0)),
        out_shape=jax.ShapeDtypeStruct((N, D_OUT), jnp.float32),
    )(a0, a1, degp_t, w, b)


@jax.jit
def kernel(x, edge_index, W_in, b_in, W_out, b_out):
    src = edge_index[0]
    dst = edge_index[1]
    pad = EPT_P * NS - E
    src_p = jnp.concatenate([src, jnp.zeros((pad,), jnp.int32)])
    dst_p = jnp.concatenate([dst, jnp.full((pad,), TRASH, jnp.int32)])
    src_p = src_p.reshape(NS, NCHUNK, CHUNK)
    dst_p = dst_p.reshape(NS, NCHUNK, CHUNK)

    degp = _deg_call(dst)
    degp_t = degp.T

    x_p = jnp.pad(x, ((0, NP - N), (0, 0)))
    s0a, s0b = _mlp1(x_p, W_in, b_in.reshape(1, D_HID), degp_t)
    s0 = jnp.stack([s0a, s0b], axis=0)

    zrows = jnp.zeros((CHUNK, DH), jnp.float32)
    _, acc = _hop_call(src_p, dst_p, degp, s0, zrows)

    y = _mlp2(acc[0, :N], acc[1, :N], degp_t[:N], W_out,
              b_out.reshape(1, D_OUT))
    return y


# speedup vs baseline: 13.9792x; 1.0043x over previous
import functools

import jax
import jax.numpy as jnp
from jax import lax
from jax.experimental import pallas as pl
from jax.experimental.pallas import tpu as pltpu
from jax.experimental.pallas import tpu_sc as plsc

N = 10000
E = 320000
D_IN = 128
D_HID = 128
D_OUT = 64
HOPS = 10

NC = 2
NS = 16
NP = 10240
RPT = NP // NS
DH = 64

EPW = E // (NC * NS)
EPT = E // NS
CHUNK = 128
NCHUNK = (EPT + CHUNK - 1) // CHUNK
EPT_P = NCHUNK * CHUNK
TRASH = NP
RCH = RPT // CHUNK

_mesh = plsc.VectorSubcoreMesh(core_axis_name="c", subcore_axis_name="s")


def _deg_body(dst_hbm, degp_out, dbuf, cnt, stage, tbuf, wacc):
    cid = lax.axis_index("c")
    sid = lax.axis_index("s")
    wid = cid * NS + sid
    zero = jnp.zeros((16,), jnp.float32)
    ones = jnp.full((16,), 1.0, jnp.float32)

    pltpu.sync_copy(dst_hbm.at[pl.ds(wid * EPW, EPW)], dbuf)

    @pl.loop(0, NP // 16)
    def _zero_cnt(i):
        cnt[pl.ds(i * 16, 16)] = zero

    @pl.loop(0, EPW // 16)
    def _count(i):
        d16 = dbuf[pl.ds(i * 16, 16)]
        plsc.addupdate_scatter(cnt, [d16], ones)

    pltpu.sync_copy(cnt, stage.at[sid])
    plsc.subcore_barrier()

    r0 = sid * RPT

    @pl.loop(0, RPT // 16)
    def _zero_w(i):
        wacc[pl.ds(i * 16, 16)] = zero

    for j in range(NS):
        pltpu.sync_copy(stage.at[j, pl.ds(r0, RPT)], tbuf)

        @pl.loop(0, RPT // 16)
        def _acc(i):
            wacc[pl.ds(i * 16, 16)] = wacc[pl.ds(i * 16, 16)] + tbuf[pl.ds(i * 16, 16)]

    pltpu.sync_copy(wacc, degp_out.at[cid, pl.ds(r0, RPT)])


_deg_call = pl.kernel(
    _deg_body,
    out_type=jax.ShapeDtypeStruct((NC, NP), jnp.float32),
    mesh=_mesh,
    scratch_types=[
        pltpu.VMEM((EPW,), jnp.int32),
        pltpu.VMEM((NP,), jnp.float32),
        pltpu.VMEM_SHARED((NS, NP), jnp.float32),
        pltpu.VMEM((RPT,), jnp.float32),
        pltpu.VMEM((RPT,), jnp.float32),
    ],
    compiler_params=pltpu.CompilerParams(needs_layout_passes=False),
)


NB = 4
NQ = 6
NU = 12


def _hop_body(idx_hbm, degp, s0, zrows, s_work, acc_out,
              t_sh, ib0, ib1, ib2, ib3, ib4, ib5,
              gb0, gb1, gb2, gb3, acc, invd, db0, db1,
              is0, is1, is2, is3, is4, is5,
              gs0, gs1, gs2, gs3, ss0, ss1, ss2, ss3,
              ws0, ws1, zs):
    ib = [ib0, ib1, ib2, ib3, ib4, ib5]
    gb = [gb0, gb1, gb2, gb3]
    isem = [is0, is1, is2, is3, is4, is5]
    gsem = [gs0, gs1, gs2, gs3]
    ssem = [ss0, ss1, ss2, ss3]
    wsem = [ws0, ws1]
    cid = lax.axis_index("c")
    sid = lax.axis_index("s")
    r0 = sid * RPT
    zero = jnp.zeros((16,), jnp.float32)
    s_view = s_work.at[cid]

    def issue_idx(j, q):
        pltpu.async_copy(idx_hbm.at[sid, j], ib[q], isem[q])

    def wait_idx(j, q):
        pltpu.make_async_copy(idx_hbm.at[sid, j], ib[q], isem[q]).wait()

    def issue_gather(b, q):
        pltpu.async_copy(s_view.at[ib[q].at[0]], gb[b], gsem[b])

    def wait_gather(b, q):
        pltpu.make_async_copy(s_view.at[ib[q].at[0]], gb[b], gsem[b]).wait()

    def issue_scatter(b, q):
        pltpu.async_copy(gb[b], t_sh.at[ib[q].at[1]], ssem[b], add=True)

    def wait_scatter(b, q):
        pltpu.make_async_copy(gb[b], t_sh.at[ib[q].at[1]], ssem[b]).wait()

    pltpu.sync_copy(degp.at[0, pl.ds(r0, RPT)], db0)
    pltpu.sync_copy(degp.at[1, pl.ds(r0, RPT)], db1)

    @pl.loop(0, RPT // 16)
    def _invd(i):
        d = jnp.maximum(db0[pl.ds(i * 16, 16)] + db1[pl.ds(i * 16, 16)], 1.0)
        invd[pl.ds(i * 16, 16)] = 1.0 / d

    for c in range(RCH):
        rg = r0 + c * CHUNK
        pltpu.sync_copy(zrows, t_sh.at[pl.ds(rg, CHUNK)])
        pltpu.sync_copy(s0.at[cid, pl.ds(rg, CHUNK)], acc.at[pl.ds(c * CHUNK, CHUNK)])
        pltpu.sync_copy(acc.at[pl.ds(c * CHUNK, CHUNK)], s_view.at[pl.ds(rg, CHUNK)])

    @pl.when(sid == NS - 1)
    def _zt():
        pltpu.sync_copy(zrows.at[pl.ds(0, 16)], t_sh.at[pl.ds(NP, 16)])

    plsc.subcore_barrier()

    def _hop(h, carry):
        issue_idx(0, 0)
        issue_idx(1, 1)
        issue_idx(2, 2)
        wait_idx(0, 0)
        issue_gather(0, 0)
        wait_idx(1, 1)
        issue_gather(1, 1)

        @pl.loop(0, NCHUNK, step=NU)
        def _blk(j0):
            for u in range(NU):
                j = j0 + u

                @pl.when(j + 3 < NCHUNK)
                def _():
                    issue_idx(j + 3, (u + 3) % NQ)

                @pl.when((j >= 2) & (j < NCHUNK + 2))
                def _():
                    wait_scatter((u + 2) % NB, (u + 4) % NQ)

                @pl.when(j + 2 < NCHUNK)
                def _():
                    wait_idx(j + 2, (u + 2) % NQ)
                    issue_gather((u + 2) % NB, (u + 2) % NQ)

                @pl.when(j < NCHUNK)
                def _():
                    wait_gather(u % NB, u % NQ)
                    issue_scatter(u % NB, u % NQ)

        plsc.subcore_barrier()

        for c in range(RCH):
            rg = r0 + c * CHUNK
            rl = c * CHUNK
            w = gb[c % 2]
            if c >= 2:
                pltpu.make_async_copy(
                    gb[c % 2],
                    s_view.at[pl.ds(r0 + (c - 2) * CHUNK, CHUNK)],
                    wsem[c % 2]).wait()
            pltpu.sync_copy(t_sh.at[pl.ds(rg, CHUNK)], w)
            pltpu.async_copy(zrows, t_sh.at[pl.ds(rg, CHUNK)], zs)

            @pl.loop(0, CHUNK)
            def _scale(r):
                inv = invd[pl.ds(rl + r, 16)][0]
                for k in range(DH // 16):
                    v = w[r, pl.ds(k * 16, 16)] * inv
                    w[r, pl.ds(k * 16, 16)] = v
                    acc[rl + r, pl.ds(k * 16, 16)] = (
                        acc[rl + r, pl.ds(k * 16, 16)] + v)

            pltpu.async_copy(w, s_view.at[pl.ds(rg, CHUNK)], wsem[c % 2])

        for c in (RCH - 2, RCH - 1):
            pltpu.make_async_copy(
                gb[c % 2], s_view.at[pl.ds(r0 + c * CHUNK, CHUNK)],
                wsem[c % 2]).wait()
        for c in range(RCH):
            pltpu.make_async_copy(
                zrows, t_sh.at[pl.ds(r0 + c * CHUNK, CHUNK)], zs).wait()
        plsc.subcore_barrier()
        return carry

    lax.fori_loop(0, HOPS, _hop, 0)

    for c in range(RCH):
        rg = r0 + c * CHUNK
        pltpu.sync_copy(acc.at[pl.ds(c * CHUNK, CHUNK)],
                        acc_out.at[cid, pl.ds(rg, CHUNK)])


_hop_call = pl.kernel(
    _hop_body,
    out_type=(
        jax.ShapeDtypeStruct((NC, NP, DH), jnp.float32),
        jax.ShapeDtypeStruct((NC, NP, DH), jnp.float32),
    ),
    mesh=_mesh,
    scratch_types=(
        [pltpu.VMEM_SHARED((NP + 16, DH), jnp.float32)]
        + [pltpu.VMEM((2, CHUNK), jnp.int32)] * NQ
        + [pltpu.VMEM((CHUNK, DH), jnp.float32)] * NB
        + [
            pltpu.VMEM((RPT, DH), jnp.float32),
            pltpu.VMEM((RPT + 16,), jnp.float32),
            pltpu.VMEM((RPT,), jnp.float32),
            pltpu.VMEM((RPT,), jnp.float32),
        ]
        + [pltpu.SemaphoreType.DMA] * (NQ + NB + NB + 3)
    ),
    compiler_params=pltpu.CompilerParams(
        needs_layout_passes=False, use_tc_tiling_on_sc=False),
)


def _mlp1_body(x_ref, w_ref, b_ref, d_ref, oa_ref, ob_ref):
    i = pl.program_id(0)
    h = jnp.dot(x_ref[...], w_ref[...], preferred_element_type=jnp.float32)
    h = jnp.maximum(h + b_ref[...], 0.0)
    d = jnp.maximum(d_ref[:, 0:1] + d_ref[:, 1:2], 1.0)
    s = h * lax.rsqrt(d)
    row = i * 640 + lax.broadcasted_iota(jnp.int32, (640, 1), 0)
    s = jnp.where(row < N, s, 0.0)
    oa_ref[...] = s[:, :DH]
    ob_ref[...] = s[:, DH:]


def _mlp1(x_p, w, b, degp_t):
    return pl.pallas_call(
        _mlp1_body,
        grid=(NP // 640,),
        in_specs=[
            pl.BlockSpec((640, D_IN), lambda i: (i, 0)),
            pl.BlockSpec((D_IN, D_HID), lambda i: (0, 0)),
            pl.BlockSpec((1, D_HID), lambda i: (0, 0)),
            pl.BlockSpec((640, 2), lambda i: (i, 0)),
        ],
        out_specs=(
            pl.BlockSpec((640, DH), lambda i: (i, 0)),
            pl.BlockSpec((640, DH), lambda i: (i, 0)),
        ),
        out_shape=(
            jax.ShapeDtypeStruct((NP, DH), jnp.float32),
            jax.ShapeDtypeStruct((NP, DH), jnp.float32),
        ),
    )(x_p, w, b, degp_t)


def _mlp2_body(a0_ref, a1_ref, d_ref, w_ref, b_ref, y_ref):
    d = jnp.maximum(d_ref[:, 0:1] + d_ref[:, 1:2], 1.0)
    scale = jnp.sqrt(d) * (1.0 / float(HOPS + 1))
    o = jnp.concatenate([a0_ref[...], a1_ref[...]], axis=1) * scale
    y = jnp.dot(o, w_ref[...], preferred_element_type=jnp.float32)
    y_ref[...] = y + b_ref[...]


def _mlp2(a0, a1, degp_t, w, b):
    blk = 1000
    return pl.pallas_call(
        _mlp2_body,
        grid=(N // blk,),
        in_specs=[
            pl.BlockSpec((blk, DH), lambda i: (i, 0)),
            pl.BlockSpec((blk, DH), lambda i: (i, 0)),
            pl.BlockSpec((blk, 2), lambda i: (i, 0)),
            pl.BlockSpec((D_HID, D_OUT), lambda i: (0, 0)),
            pl.BlockSpec((1, D_OUT), lambda i: (0, 0)),
        ],
        out_specs=pl.BlockSpec((blk, D_OUT), lambda i: (i, 0)),
        out_shape=jax.ShapeDtypeStruct((N, D_OUT), jnp.float32),
    )(a0, a1, degp_t, w, b)


@jax.jit
def kernel(x, edge_index, W_in, b_in, W_out, b_out):
    src = edge_index[0]
    dst = edge_index[1]
    pad = EPT_P - EPT
    src_p = jnp.concatenate(
        [src.reshape(NS, EPT), jnp.zeros((NS, pad), jnp.int32)], axis=1
    ).reshape(NS, NCHUNK, 1, CHUNK)
    dst_p = jnp.concatenate(
        [dst.reshape(NS, EPT), jnp.full((NS, pad), TRASH, jnp.int32)], axis=1
    ).reshape(NS, NCHUNK, 1, CHUNK)
    idx_p = jnp.concatenate([src_p, dst_p], axis=2)

    degp = _deg_call(dst)
    degp_t = degp.T

    x_p = jnp.pad(x, ((0, NP - N), (0, 0)))
    s0a, s0b = _mlp1(x_p, W_in, b_in.reshape(1, D_HID), degp_t)
    s0 = jnp.stack([s0a, s0b], axis=0)

    zrows = jnp.zeros((CHUNK, DH), jnp.float32)
    _, acc = _hop_call(idx_p, degp, s0, zrows)

    y = _mlp2(acc[0, :N], acc[1, :N], degp_t[:N], W_out,
              b_out.reshape(1, D_OUT))
    return y
